# baseline (speedup 1.0000x reference)
import functools

import jax
import jax.numpy as jnp
from jax import lax
from jax.experimental import pallas as pl
from jax.experimental.pallas import tpu as pltpu
from jax.experimental.pallas import tpu_sc as plsc

B = 16384
L = 50
EMBED = 64
NC, NS = 2, 16
NW = NC * NS
B_PER_W = B // NW
CHUNK = 128
KPL = B_PER_W // CHUNK
NBUF = 8
NGROUP = L * KPL // NBUF

_mesh = plsc.VectorSubcoreMesh(core_axis_name="c", subcore_axis_name="s")


def _iota16(off):
    return jnp.arange(16, dtype=jnp.int32) + off


@functools.partial(
    pl.kernel,
    mesh=_mesh,
    out_type=jax.ShapeDtypeStruct((L, 8, B // CHUNK, 8, CHUNK), jnp.float32),
    scratch_types=[
        pltpu.VMEM((L, B_PER_W), jnp.int32),
        pltpu.VMEM((NBUF, CHUNK, EMBED), jnp.float32),
        pltpu.VMEM((2, 8, 8, CHUNK), jnp.float32),
        pltpu.SemaphoreType.DMA((NBUF,)),
        pltpu.SemaphoreType.DMA((2,)),
    ],
    compiler_params=pltpu.CompilerParams(
        use_tc_tiling_on_sc=False, needs_layout_passes=False
    ),
)
def _gather_all(idxt_hbm, table_hbm, out_hbm, idx_v, rows_v, tbuf, semg, semo):
    wid = lax.axis_index("s") * NC + lax.axis_index("c")
    base = wid * B_PER_W

    pltpu.sync_copy(idxt_hbm.at[:, pl.ds(base, B_PER_W)], idx_v)

    m_iotas = [_iota16(16 * j) for j in range(8)]

    def group(g, carry):
        for b in range(NBUF):
            l = 2 * g + b // KPL
            k = b % KPL
            pltpu.async_copy(
                table_hbm.at[idx_v.at[l, pl.ds(k * CHUNK, CHUNK)]],
                rows_v.at[b],
                semg.at[b],
            )
        for b in range(NBUF):
            l = 2 * g + b // KPL
            k = b % KPL
            sl = b % 2
            pltpu.make_async_copy(
                table_hbm.at[idx_v.at[l, pl.ds(k * CHUNK, CHUNK)]],
                rows_v.at[b],
                semg.at[b],
            ).wait()
            if b >= 2:
                pltpu.make_async_copy(
                    tbuf.at[sl], out_hbm.at[0, :, 0], semo.at[sl]
                ).wait()
            else:
                @pl.when(g > 0)
                def _wait_store():
                    pltpu.make_async_copy(
                        tbuf.at[sl], out_hbm.at[0, :, 0], semo.at[sl]
                    ).wait()

            def crow(c, c2):
                q = c // 8
                s = c - q * 8
                cs = _iota16(0) * 0 + c
                for j in range(8):
                    v = plsc.load_gather(rows_v.at[b], [m_iotas[j], cs])
                    tbuf[sl, q, s, pl.ds(16 * j, 16)] = v
                return c2

            lax.fori_loop(0, EMBED, crow, 0)

            pltpu.async_copy(
                tbuf.at[sl],
                out_hbm.at[l, :, wid * KPL + k],
                semo.at[sl],
            )
        return carry

    lax.fori_loop(0, NGROUP, group, 0)

    for sl in range(2):
        pltpu.make_async_copy(
            tbuf.at[sl], out_hbm.at[0, :, 0], semo.at[sl]
        ).wait()


def kernel(input, table):
    p5 = _gather_all(input.T, table)
    return p5.transpose(2, 4, 0, 1, 3).reshape(B, L, EMBED)


# speedup vs baseline: 1.2053x; 1.2053x over previous
import functools

import jax
import jax.numpy as jnp
from jax import lax
from jax.experimental import pallas as pl
from jax.experimental.pallas import tpu as pltpu
from jax.experimental.pallas import tpu_sc as plsc

B = 16384
L = 50
EMBED = 64
NC, NS = 2, 16
NW = NC * NS
B_PER_W = B // NW
CHUNK = 128
KPL = B_PER_W // CHUNK
NBUF = 8
NGROUP = L * KPL // NBUF

_mesh = plsc.VectorSubcoreMesh(core_axis_name="c", subcore_axis_name="s")


def _iota16(off):
    return jnp.arange(16, dtype=jnp.int32) + off


@functools.partial(
    pl.kernel,
    mesh=_mesh,
    out_type=jax.ShapeDtypeStruct((L, 8, B // CHUNK, 8, CHUNK), jnp.float32),
    scratch_types=[
        pltpu.VMEM((L, B_PER_W), jnp.int32),
        pltpu.VMEM((NBUF, CHUNK, EMBED), jnp.float32),
        pltpu.VMEM((2, 8, 8, CHUNK), jnp.float32),
        pltpu.SemaphoreType.DMA((NBUF,)),
        pltpu.SemaphoreType.DMA((2,)),
    ],
    compiler_params=pltpu.CompilerParams(
        use_tc_tiling_on_sc=False, needs_layout_passes=False
    ),
)
def _gather_all(idxt_hbm, table_hbm, out_hbm, idx_v, rows_v, tbuf, semg, semo):
    wid = lax.axis_index("s") * NC + lax.axis_index("c")
    base = wid * B_PER_W

    pltpu.sync_copy(idxt_hbm.at[:, pl.ds(base, B_PER_W)], idx_v)

    m_iotas = [_iota16(16 * j) for j in range(8)]

    def group(g, carry):
        for b in range(NBUF):
            l = 2 * g + b // KPL
            k = b % KPL
            pltpu.async_copy(
                table_hbm.at[idx_v.at[l, pl.ds(k * CHUNK, CHUNK)]],
                rows_v.at[b],
                semg.at[b],
            )
        for b in range(NBUF):
            l = 2 * g + b // KPL
            k = b % KPL
            sl = b % 2
            pltpu.make_async_copy(
                table_hbm.at[idx_v.at[l, pl.ds(k * CHUNK, CHUNK)]],
                rows_v.at[b],
                semg.at[b],
            ).wait()
            if b >= 2:
                pltpu.make_async_copy(
                    tbuf.at[sl], out_hbm.at[0, :, 0], semo.at[sl]
                ).wait()
            else:
                @pl.when(g > 0)
                def _wait_store():
                    pltpu.make_async_copy(
                        tbuf.at[sl], out_hbm.at[0, :, 0], semo.at[sl]
                    ).wait()

            @plsc.parallel_loop(0, EMBED, unroll=4)
            def crow(c):
                q = c // 8
                s = c - q * 8
                cs = _iota16(0) * 0 + c
                vs = [
                    plsc.load_gather(rows_v.at[b], [m_iotas[j], cs])
                    for j in range(8)
                ]
                for j in range(8):
                    tbuf[sl, q, s, pl.ds(16 * j, 16)] = vs[j]

            pltpu.async_copy(
                tbuf.at[sl],
                out_hbm.at[l, :, wid * KPL + k],
                semo.at[sl],
            )
        return carry

    lax.fori_loop(0, NGROUP, group, 0)

    for sl in range(2):
        pltpu.make_async_copy(
            tbuf.at[sl], out_hbm.at[0, :, 0], semo.at[sl]
        ).wait()


def kernel(input, table):
    p5 = _gather_all(input.T, table)
    return p5.transpose(2, 4, 0, 1, 3).reshape(B, L, EMBED)


# speedup vs baseline: 1.5649x; 1.2984x over previous
import functools

import jax
import jax.numpy as jnp
from jax import lax
from jax.experimental import pallas as pl
from jax.experimental.pallas import tpu as pltpu
from jax.experimental.pallas import tpu_sc as plsc

B = 16384
L = 50
EMBED = 64
NC, NS = 2, 16
NW = NC * NS
B_PER_W = B // NW
CHUNK = 128
KPL = B_PER_W // CHUNK
NBUF = 8
NGROUP = L * KPL // NBUF

_mesh = plsc.VectorSubcoreMesh(core_axis_name="c", subcore_axis_name="s")


@functools.partial(
    pl.kernel,
    mesh=_mesh,
    out_type=jax.ShapeDtypeStruct((L, B, EMBED), jnp.float32),
    scratch_types=[
        pltpu.VMEM((L, B_PER_W), jnp.int32),
        pltpu.VMEM((NBUF, CHUNK, EMBED), jnp.float32),
        pltpu.SemaphoreType.DMA((NBUF,)),
        pltpu.SemaphoreType.DMA((NBUF,)),
    ],
    compiler_params=pltpu.CompilerParams(use_tc_tiling_on_sc=False),
)
def _gather_all(idxt_hbm, table_hbm, out_hbm, idx_v, rows_v, semg, sems):
    wid = lax.axis_index("s") * NC + lax.axis_index("c")
    base = wid * B_PER_W

    pltpu.sync_copy(idxt_hbm.at[:, pl.ds(base, B_PER_W)], idx_v)

    def group(g, carry):
        for b in range(NBUF):
            l = 2 * g + b // KPL
            k = b % KPL

            @pl.when(g > 0)
            def _wait_store():
                pltpu.make_async_copy(
                    rows_v.at[b], out_hbm.at[0, pl.ds(0, CHUNK)], sems.at[b]
                ).wait()

            pltpu.async_copy(
                table_hbm.at[idx_v.at[l, pl.ds(k * CHUNK, CHUNK)]],
                rows_v.at[b],
                semg.at[b],
            )
        for b in range(NBUF):
            l = 2 * g + b // KPL
            k = b % KPL
            pltpu.make_async_copy(
                table_hbm.at[idx_v.at[l, pl.ds(k * CHUNK, CHUNK)]],
                rows_v.at[b],
                semg.at[b],
            ).wait()
            pltpu.async_copy(
                rows_v.at[b],
                out_hbm.at[l, pl.ds(base + k * CHUNK, CHUNK)],
                sems.at[b],
            )
        return carry

    lax.fori_loop(0, NGROUP, group, 0)

    for b in range(NBUF):
        pltpu.make_async_copy(
            rows_v.at[b], out_hbm.at[0, pl.ds(0, CHUNK)], sems.at[b]
        ).wait()


def kernel(input, table):
    return _gather_all(input.T, table).transpose(1, 0, 2)
